# baseline (speedup 1.0000x reference)
import functools

import jax
import jax.numpy as jnp
from jax.experimental import pallas as pl
from jax.experimental.pallas import tpu as pltpu

_VMEM_BUDGET = 30 * 1024 * 1024
_VMEM_LIMIT_BYTES = 48 * 1024 * 1024


def _se_kernel(x_ref, w1_ref, w2_ref, o_ref, *, inv_hw):
    x = x_ref[...]
    pooled = jnp.sum(x, axis=2, dtype=jnp.float32) * inv_hw
    h = jnp.maximum(
        jnp.dot(pooled, w1_ref[...], preferred_element_type=jnp.float32), 0.0)
    gate = jax.nn.sigmoid(
        jnp.dot(h, w2_ref[...], preferred_element_type=jnp.float32))
    o_ref[...] = x * gate.astype(o_ref.dtype)[:, :, None]


def _pick_batch_tile(b, block_bytes_per_batch, budget_bytes):
    cap = 1 if b == 1 else b // 2
    best = 1
    for bt in range(1, cap + 1):
        if b % bt == 0 and 4 * bt * block_bytes_per_batch <= budget_bytes:
            best = bt
    return best


def kernel(x_nchw, w1, w2):
    B, C, H, W = x_nchw.shape
    HW = H * W
    Cr = w1.shape[1]

    x3 = x_nchw.reshape(B, C, HW)

    itemsize = jnp.dtype(x_nchw.dtype).itemsize
    hw_padded = -(-HW // 128) * 128
    bt = _pick_batch_tile(B, C * hw_padded * itemsize, _VMEM_BUDGET)

    body = functools.partial(_se_kernel, inv_hw=1.0 / float(HW))
    out = pl.pallas_call(
        body,
        out_shape=jax.ShapeDtypeStruct((B, C, HW), x_nchw.dtype),
        grid=(B // bt,),
        in_specs=[
            pl.BlockSpec((bt, C, HW), lambda b: (b, 0, 0)),
            pl.BlockSpec((C, Cr), lambda b: (0, 0)),
            pl.BlockSpec((Cr, C), lambda b: (0, 0)),
        ],
        out_specs=pl.BlockSpec((bt, C, HW), lambda b: (b, 0, 0)),
        compiler_params=pltpu.CompilerParams(
            dimension_semantics=("parallel",),
            vmem_limit_bytes=_VMEM_LIMIT_BYTES,
        ),
    )(x3, w1, w2)

    return out.reshape(B, C, H, W)
